# Initial kernel scaffold; baseline (speedup 1.0000x reference)
import jax
import jax.numpy as jnp
from jax.experimental import pallas as pl


def kernel(hidden_states, router_W, router_b, expert_W, expert_b):
    raise NotImplementedError("write your pallas kernel here")


# speedup vs baseline: 2.6993x; 2.6993x over previous
import functools

import jax
import jax.numpy as jnp
from jax import lax
from jax.experimental import pallas as pl
from jax.experimental.pallas import tpu as pltpu
from jax.experimental.pallas import tpu_sc as plsc

E = 64
H = 768
S = 2048
A = 2 * S
T = 128
NT = A // T
G = NT + E

NC = 2
NS = 16
NW = NC * NS


def _router_body(x_ref, w_ref, b_ref, out_ref):
    x = x_ref[...]
    w = w_ref[...]
    logits = lax.dot_general(x, w, (((1,), (1,)), ((), ())),
                             preferred_element_type=jnp.float32)
    logits = logits + b_ref[...]
    ii = lax.broadcasted_iota(jnp.int32, logits.shape, 1)
    m1 = jnp.max(logits, axis=1, keepdims=True)
    a1 = jnp.min(jnp.where(logits == m1, ii, E), axis=1, keepdims=True)
    l2 = jnp.where(ii == a1, -jnp.inf, logits)
    m2 = jnp.max(l2, axis=1, keepdims=True)
    a2 = jnp.min(jnp.where(l2 == m2, ii, E), axis=1, keepdims=True)
    lane = lax.broadcasted_iota(jnp.int32, (S, 128), 1)
    out_ref[...] = jnp.where(lane == 0, a1, jnp.where(lane == 1, a2, 0))


def _route(x2d, router_W, router_b):
    return pl.pallas_call(
        _router_body,
        out_shape=jax.ShapeDtypeStruct((S, 128), jnp.int32),
    )(x2d, router_W, router_b.reshape(1, E))


def _gather_body(x_hbm, idx_hbm, out_hbm, idx_v, rows_v, sem):
    wid = lax.axis_index("s") * NC + lax.axis_index("c")
    rows = A // NW
    base = wid * rows
    pltpu.sync_copy(idx_hbm.at[pl.ds(base, rows)], idx_v)
    pltpu.async_copy(x_hbm.at[idx_v], rows_v, sem).wait()
    pltpu.sync_copy(rows_v, out_hbm.at[pl.ds(base, rows)])


def _gather_sorted(x2d, sorted_tok):
    mesh = plsc.VectorSubcoreMesh(core_axis_name="c", subcore_axis_name="s",
                                  num_cores=NC, num_subcores=NS)
    rows = A // NW
    return pl.kernel(
        _gather_body,
        out_type=jax.ShapeDtypeStruct((A, H), jnp.float32),
        mesh=mesh,
        scratch_types=[
            pltpu.VMEM((rows,), jnp.int32),
            pltpu.VMEM((rows, H), jnp.float32),
            pltpu.SemaphoreType.DMA,
        ],
    )(x2d, sorted_tok)


def _gmm_body(pm_ref, pe_ref, off_ref, np_ref, xs_ref, w_ref, b_ref, out_ref):
    i = pl.program_id(0)
    first = jnp.logical_or(i == 0, pm_ref[jnp.maximum(i - 1, 0)] != pm_ref[i])

    @pl.when(first)
    def _():
        out_ref[...] = jnp.zeros_like(out_ref)

    @pl.when(i < np_ref[0])
    def _():
        e = pe_ref[i]
        x = xs_ref[...]
        w = w_ref[0]
        y = lax.dot_general(x, w, (((1,), (1,)), ((), ())),
                            preferred_element_type=jnp.float32)
        y = y + b_ref[0]
        r = lax.broadcasted_iota(jnp.int32, (T, 1), 0) + pm_ref[i] * T
        mask = (r >= off_ref[e]) & (r < off_ref[e + 1])
        out_ref[...] += jnp.where(mask, y, 0.0)


def _grouped_matmul(xs, expert_W, expert_b, pair_m, pair_e, off, npairs):
    grid_spec = pltpu.PrefetchScalarGridSpec(
        num_scalar_prefetch=4,
        grid=(G,),
        in_specs=[
            pl.BlockSpec((T, H), lambda i, pm, pe, off, np_: (pm[i], 0)),
            pl.BlockSpec((1, H, H), lambda i, pm, pe, off, np_: (pe[i], 0, 0)),
            pl.BlockSpec((1, 1, H), lambda i, pm, pe, off, np_: (pe[i], 0, 0)),
        ],
        out_specs=pl.BlockSpec((T, H), lambda i, pm, pe, off, np_: (pm[i], 0)),
    )
    return pl.pallas_call(
        _gmm_body,
        grid_spec=grid_spec,
        out_shape=jax.ShapeDtypeStruct((A, H), jnp.float32),
    )(pair_m, pair_e, off, npairs, xs, expert_W, expert_b.reshape(E, 1, H))


def _combine_body(y_hbm, p1_hbm, p2_hbm, out_hbm,
                  i1_v, i2_v, y1_v, y2_v, sem1, sem2):
    wid = lax.axis_index("s") * NC + lax.axis_index("c")
    rows = S // NW
    base = wid * rows
    pltpu.sync_copy(p1_hbm.at[pl.ds(base, rows)], i1_v)
    pltpu.sync_copy(p2_hbm.at[pl.ds(base, rows)], i2_v)
    c1 = pltpu.async_copy(y_hbm.at[i1_v], y1_v, sem1)
    c2 = pltpu.async_copy(y_hbm.at[i2_v], y2_v, sem2)
    c1.wait()
    c2.wait()

    def add16(i, _):
        r = i // (H // 16)
        c = (i % (H // 16)) * 16
        y1_v[r, pl.ds(c, 16)] = y1_v[r, pl.ds(c, 16)] + y2_v[r, pl.ds(c, 16)]
        return 0

    lax.fori_loop(0, rows * (H // 16), add16, 0)
    pltpu.sync_copy(y1_v, out_hbm.at[pl.ds(base, rows)])


def _combine(ys, p1, p2):
    mesh = plsc.VectorSubcoreMesh(core_axis_name="c", subcore_axis_name="s",
                                  num_cores=NC, num_subcores=NS)
    rows = S // NW
    return pl.kernel(
        _combine_body,
        out_type=jax.ShapeDtypeStruct((S, H), jnp.float32),
        mesh=mesh,
        scratch_types=[
            pltpu.VMEM((rows,), jnp.int32),
            pltpu.VMEM((rows,), jnp.int32),
            pltpu.VMEM((rows, H), jnp.float32),
            pltpu.VMEM((rows, H), jnp.float32),
            pltpu.SemaphoreType.DMA,
            pltpu.SemaphoreType.DMA,
        ],
    )(ys, p1, p2)


@jax.jit
def kernel(hidden_states, router_W, router_b, expert_W, expert_b):
    x2d = hidden_states.reshape(S, H)

    routed = _route(x2d, router_W, router_b)
    a1 = routed[:, 0]
    a2 = routed[:, 1]

    e_flat = jnp.concatenate([a1, a2])
    tok = jnp.arange(S, dtype=jnp.int32)
    tok_flat = jnp.concatenate([tok, tok])
    order = jnp.argsort(e_flat, stable=True)
    sorted_tok = tok_flat[order].astype(jnp.int32)
    inv = jnp.zeros((A,), jnp.int32).at[order].set(
        jnp.arange(A, dtype=jnp.int32))
    p1 = inv[:S]
    p2 = inv[S:]

    counts = jnp.zeros((E,), jnp.int32).at[e_flat].add(1)
    off = jnp.concatenate(
        [jnp.zeros((1,), jnp.int32), jnp.cumsum(counts).astype(jnp.int32)])

    jT = jnp.arange(NT, dtype=jnp.int32) * T
    hit = (off[None, :-1] < (jT[:, None] + T)) & (off[None, 1:] > jT[:, None])
    hitf = hit.reshape(-1)
    pos = jnp.cumsum(hitf.astype(jnp.int32)) - 1
    num_pairs = jnp.sum(hitf.astype(jnp.int32))
    jv = jnp.repeat(jnp.arange(NT, dtype=jnp.int32), E)
    ev = jnp.tile(jnp.arange(E, dtype=jnp.int32), NT)
    scat = jnp.where(hitf, pos, G)
    pair_m = jnp.zeros((G,), jnp.int32).at[scat].set(jv, mode="drop")
    pair_e = jnp.zeros((G,), jnp.int32).at[scat].set(ev, mode="drop")
    last = jnp.maximum(num_pairs - 1, 0)
    gi = jnp.arange(G, dtype=jnp.int32)
    pair_m = jnp.where(gi < num_pairs, pair_m, pair_m[last])
    pair_e = jnp.where(gi < num_pairs, pair_e, pair_e[last])
    npairs = num_pairs.reshape(1)

    xs = _gather_sorted(x2d, sorted_tok)
    ys = _grouped_matmul(xs, expert_W, expert_b,
                         pair_m, pair_e, off, npairs)
    out2d = _combine(ys, p1, p2)
    return out2d.reshape(1, S, H)


# speedup vs baseline: 2.7246x; 1.0094x over previous
import functools

import jax
import jax.numpy as jnp
from jax import lax
from jax.experimental import pallas as pl
from jax.experimental.pallas import tpu as pltpu
from jax.experimental.pallas import tpu_sc as plsc

E = 64
H = 768
S = 2048
A = 2 * S
T = 128
NT = A // T
G = NT + E

NC = 2
NS = 16
NW = NC * NS


def _router_body(x_ref, w_ref, b_ref, pos_ref, off_ref):
    x = x_ref[...]
    w = w_ref[...]
    logits = lax.dot_general(x, w, (((1,), (1,)), ((), ())),
                             preferred_element_type=jnp.float32)
    logits = logits + b_ref[...]
    ii = lax.broadcasted_iota(jnp.int32, logits.shape, 1)
    m1 = jnp.max(logits, axis=1, keepdims=True)
    a1 = jnp.min(jnp.where(logits == m1, ii, E), axis=1, keepdims=True)
    l2 = jnp.where(ii == a1, -jnp.inf, logits)
    m2 = jnp.max(l2, axis=1, keepdims=True)
    a2 = jnp.min(jnp.where(l2 == m2, ii, E), axis=1, keepdims=True)
    acol = jnp.concatenate([a1, a2], axis=0)

    ohs = []
    counts = jnp.zeros((1, E), jnp.float32)
    for j in range(A // T):
        ac = lax.slice(acol, (j * T, 0), ((j + 1) * T, 1))
        oh = (ac == lax.broadcasted_iota(jnp.int32, (T, E), 1)
              ).astype(jnp.float32)
        ohs.append(oh)
        counts = counts + jnp.sum(oh, axis=0, keepdims=True)
    iu = lax.broadcasted_iota(jnp.int32, (E, E), 0)
    ustrict = (iu < lax.broadcasted_iota(jnp.int32, (E, E), 1)
               ).astype(jnp.float32)
    off_row = lax.dot_general(counts, ustrict, (((1,), (0,)), ((), ())),
                              precision=lax.Precision.HIGHEST,
                              preferred_element_type=jnp.float32)

    il = lax.broadcasted_iota(jnp.int32, (T, T), 0)
    lstrict = (il > lax.broadcasted_iota(jnp.int32, (T, T), 1)
               ).astype(jnp.float32)
    eye = (il == lax.broadcasted_iota(jnp.int32, (T, T), 1)
           ).astype(jnp.float32)

    run = jnp.zeros((1, E), jnp.float32)
    for j in range(A // T):
        oh = ohs[j]
        pre = lax.dot_general(lstrict, oh, (((1,), (0,)), ((), ())),
                              precision=lax.Precision.HIGHEST,
                              preferred_element_type=jnp.float32) + run
        rank = jnp.sum(pre * oh, axis=1, keepdims=True)
        start = jnp.sum(off_row * oh, axis=1, keepdims=True)
        pos_col = rank + start
        pos_row = lax.dot_general(pos_col, eye, (((0,), (0,)), ((), ())),
                                  precision=lax.Precision.HIGHEST,
                                  preferred_element_type=jnp.float32)
        pos_ref[pl.ds(j, 1), :] = pos_row.astype(jnp.int32)
        run = run + jnp.sum(oh, axis=0, keepdims=True)
    off_ref[...] = off_row.astype(jnp.int32)


def _route(x2d, router_W, router_b):
    return pl.pallas_call(
        _router_body,
        out_shape=[
            jax.ShapeDtypeStruct((A // T, T), jnp.int32),
            jax.ShapeDtypeStruct((1, E), jnp.int32),
        ],
    )(x2d, router_W, router_b.reshape(1, E))


def _scatter_body(x_hbm, idx_hbm, out_hbm, idx_v, rows_v, sem):
    wid = lax.axis_index("s") * NC + lax.axis_index("c")
    rows = A // NW
    base = wid * rows
    pltpu.sync_copy(idx_hbm.at[pl.ds(base, rows)], idx_v)
    pltpu.async_copy(x_hbm.at[idx_v], rows_v, sem).wait()
    pltpu.sync_copy(rows_v, out_hbm.at[pl.ds(base, rows)])


def _scatter_sorted(x2d, pos):
    mesh = plsc.VectorSubcoreMesh(core_axis_name="c", subcore_axis_name="s",
                                  num_cores=NC, num_subcores=NS)
    rows = A // NW
    return pl.kernel(
        _scatter_body,
        out_type=jax.ShapeDtypeStruct((A, H), jnp.float32),
        mesh=mesh,
        scratch_types=[
            pltpu.VMEM((rows,), jnp.int32),
            pltpu.VMEM((rows, H), jnp.float32),
            pltpu.SemaphoreType.DMA,
        ],
    )(x2d, pos)


def _gmm_body(pm_ref, pe_ref, off_ref, np_ref, xs_ref, w_ref, b_ref, out_ref):
    i = pl.program_id(0)
    first = jnp.logical_or(i == 0, pm_ref[jnp.maximum(i - 1, 0)] != pm_ref[i])

    @pl.when(first)
    def _():
        out_ref[...] = jnp.zeros_like(out_ref)

    @pl.when(i < np_ref[0])
    def _():
        e = pe_ref[i]
        x = xs_ref[...]
        w = w_ref[0]
        y = lax.dot_general(x, w, (((1,), (1,)), ((), ())),
                            preferred_element_type=jnp.float32)
        y = y + b_ref[0]
        r = lax.broadcasted_iota(jnp.int32, (T, 1), 0) + pm_ref[i] * T
        mask = (r >= off_ref[e]) & (r < off_ref[e + 1])
        out_ref[...] += jnp.where(mask, y, 0.0)


def _grouped_matmul(xs, expert_W, expert_b, pair_m, pair_e, off, npairs):
    grid_spec = pltpu.PrefetchScalarGridSpec(
        num_scalar_prefetch=4,
        grid=(G,),
        in_specs=[
            pl.BlockSpec((T, H), lambda i, pm, pe, off, np_: (pm[i], 0)),
            pl.BlockSpec((1, H, H), lambda i, pm, pe, off, np_: (pe[i], 0, 0)),
            pl.BlockSpec((1, 1, H), lambda i, pm, pe, off, np_: (pe[i], 0, 0)),
        ],
        out_specs=pl.BlockSpec((T, H), lambda i, pm, pe, off, np_: (pm[i], 0)),
    )
    return pl.pallas_call(
        _gmm_body,
        grid_spec=grid_spec,
        out_shape=jax.ShapeDtypeStruct((A, H), jnp.float32),
    )(pair_m, pair_e, off, npairs, xs, expert_W, expert_b.reshape(E, 1, H))


def _combine_body(y_hbm, p1_hbm, p2_hbm, out_hbm,
                  i1_v, i2_v, y1_v, y2_v, sem1, sem2):
    wid = lax.axis_index("s") * NC + lax.axis_index("c")
    rows = S // NW
    base = wid * rows
    pltpu.sync_copy(p1_hbm.at[pl.ds(base, rows)], i1_v)
    pltpu.sync_copy(p2_hbm.at[pl.ds(base, rows)], i2_v)
    c1 = pltpu.async_copy(y_hbm.at[i1_v], y1_v, sem1)
    c2 = pltpu.async_copy(y_hbm.at[i2_v], y2_v, sem2)
    c1.wait()
    c2.wait()

    def add16(i, _):
        r = i // (H // 16)
        c = (i % (H // 16)) * 16
        y1_v[r, pl.ds(c, 16)] = y1_v[r, pl.ds(c, 16)] + y2_v[r, pl.ds(c, 16)]
        return 0

    lax.fori_loop(0, rows * (H // 16), add16, 0)
    pltpu.sync_copy(y1_v, out_hbm.at[pl.ds(base, rows)])


def _combine(ys, p1, p2):
    mesh = plsc.VectorSubcoreMesh(core_axis_name="c", subcore_axis_name="s",
                                  num_cores=NC, num_subcores=NS)
    rows = S // NW
    return pl.kernel(
        _combine_body,
        out_type=jax.ShapeDtypeStruct((S, H), jnp.float32),
        mesh=mesh,
        scratch_types=[
            pltpu.VMEM((rows,), jnp.int32),
            pltpu.VMEM((rows,), jnp.int32),
            pltpu.VMEM((rows, H), jnp.float32),
            pltpu.VMEM((rows, H), jnp.float32),
            pltpu.SemaphoreType.DMA,
            pltpu.SemaphoreType.DMA,
        ],
    )(ys, p1, p2)


@jax.jit
def kernel(hidden_states, router_W, router_b, expert_W, expert_b):
    x2d = hidden_states.reshape(S, H)

    posr, offr = _route(x2d, router_W, router_b)
    pos = posr.reshape(A)
    p1 = pos[:S]
    p2 = pos[S:]
    off = jnp.concatenate(
        [offr.reshape(E), jnp.full((1,), A, jnp.int32)])

    jT = jnp.arange(NT, dtype=jnp.int32) * T
    hit = (off[None, :-1] < (jT[:, None] + T)) & (off[None, 1:] > jT[:, None])
    hitf = hit.reshape(-1)
    prank = jnp.cumsum(hitf.astype(jnp.int32)) - 1
    num_pairs = jnp.sum(hitf.astype(jnp.int32))
    jv = jnp.repeat(jnp.arange(NT, dtype=jnp.int32), E)
    ev = jnp.tile(jnp.arange(E, dtype=jnp.int32), NT)
    scat = jnp.where(hitf, prank, G)
    pair_m = jnp.zeros((G,), jnp.int32).at[scat].set(jv, mode="drop")
    pair_e = jnp.zeros((G,), jnp.int32).at[scat].set(ev, mode="drop")
    last = jnp.maximum(num_pairs - 1, 0)
    gi = jnp.arange(G, dtype=jnp.int32)
    pair_m = jnp.where(gi < num_pairs, pair_m, pair_m[last])
    pair_e = jnp.where(gi < num_pairs, pair_e, pair_e[last])
    npairs = num_pairs.reshape(1)

    tok = jnp.arange(S, dtype=jnp.int32)
    sorted_tok = jnp.zeros((A,), jnp.int32).at[pos].set(
        jnp.concatenate([tok, tok]))
    xs = _scatter_sorted(x2d, sorted_tok)
    ys = _grouped_matmul(xs, expert_W, expert_b,
                         pair_m, pair_e, off, npairs)
    out2d = _combine(ys, p1, p2)
    return out2d.reshape(1, S, H)


# speedup vs baseline: 3.1711x; 1.1639x over previous
import functools

import jax
import jax.numpy as jnp
from jax import lax
from jax.experimental import pallas as pl
from jax.experimental.pallas import tpu as pltpu
from jax.experimental.pallas import tpu_sc as plsc

E = 64
H = 768
S = 2048
A = 2 * S
T = 128
NT = A // T
G = NT + E

NC = 2
NS = 16
NW = NC * NS


def _router_body(x_ref, w_ref, b_ref, pos_ref, off_ref):
    x = x_ref[...]
    w = w_ref[...]
    logits = lax.dot_general(x, w, (((1,), (1,)), ((), ())),
                             preferred_element_type=jnp.float32)
    logits = logits + b_ref[...]
    ii = lax.broadcasted_iota(jnp.int32, logits.shape, 1)
    m1 = jnp.max(logits, axis=1, keepdims=True)
    a1 = jnp.min(jnp.where(logits == m1, ii, E), axis=1, keepdims=True)
    l2 = jnp.where(ii == a1, -jnp.inf, logits)
    m2 = jnp.max(l2, axis=1, keepdims=True)
    a2 = jnp.min(jnp.where(l2 == m2, ii, E), axis=1, keepdims=True)
    acol = jnp.concatenate([a1, a2], axis=0)

    ohs = []
    counts = jnp.zeros((1, E), jnp.float32)
    for j in range(A // T):
        ac = lax.slice(acol, (j * T, 0), ((j + 1) * T, 1))
        oh = (ac == lax.broadcasted_iota(jnp.int32, (T, E), 1)
              ).astype(jnp.float32)
        ohs.append(oh)
        counts = counts + jnp.sum(oh, axis=0, keepdims=True)
    iu = lax.broadcasted_iota(jnp.int32, (E, E), 0)
    ustrict = (iu < lax.broadcasted_iota(jnp.int32, (E, E), 1)
               ).astype(jnp.float32)
    off_row = lax.dot_general(counts, ustrict, (((1,), (0,)), ((), ())),
                              precision=lax.Precision.HIGHEST,
                              preferred_element_type=jnp.float32)

    il = lax.broadcasted_iota(jnp.int32, (T, T), 0)
    lstrict = (il > lax.broadcasted_iota(jnp.int32, (T, T), 1)
               ).astype(jnp.float32)
    eye = (il == lax.broadcasted_iota(jnp.int32, (T, T), 1)
           ).astype(jnp.float32)

    run = jnp.zeros((1, E), jnp.float32)
    for j in range(A // T):
        oh = ohs[j]
        pre = lax.dot_general(lstrict, oh, (((1,), (0,)), ((), ())),
                              precision=lax.Precision.HIGHEST,
                              preferred_element_type=jnp.float32) + run
        rank = jnp.sum(pre * oh, axis=1, keepdims=True)
        start = jnp.sum(off_row * oh, axis=1, keepdims=True)
        pos_col = rank + start
        pos_row = lax.dot_general(pos_col, eye, (((0,), (0,)), ((), ())),
                                  precision=lax.Precision.HIGHEST,
                                  preferred_element_type=jnp.float32)
        pos_ref[pl.ds(j, 1), :] = pos_row.astype(jnp.int32)
        run = run + jnp.sum(oh, axis=0, keepdims=True)
    off_ref[...] = off_row.astype(jnp.int32)


def _route(x2d, router_W, router_b):
    return pl.pallas_call(
        _router_body,
        out_shape=[
            jax.ShapeDtypeStruct((A // T, T), jnp.int32),
            jax.ShapeDtypeStruct((1, E), jnp.int32),
        ],
    )(x2d, router_W, router_b.reshape(1, E))


def _scatter_body(x_hbm, idx_hbm, out_hbm, idx_v, rows_v, sem):
    wid = lax.axis_index("s") * NC + lax.axis_index("c")
    rows = A // NW
    base = wid * rows
    pltpu.sync_copy(idx_hbm.at[pl.ds(base, rows)], idx_v)
    pltpu.async_copy(x_hbm.at[idx_v], rows_v, sem).wait()
    pltpu.sync_copy(rows_v, out_hbm.at[pl.ds(base, rows)])


def _scatter_sorted(x2d, pos):
    mesh = plsc.VectorSubcoreMesh(core_axis_name="c", subcore_axis_name="s",
                                  num_cores=NC, num_subcores=NS)
    rows = A // NW
    return pl.kernel(
        _scatter_body,
        out_type=jax.ShapeDtypeStruct((A, H), jnp.float32),
        mesh=mesh,
        scratch_types=[
            pltpu.VMEM((rows,), jnp.int32),
            pltpu.VMEM((rows, H), jnp.float32),
            pltpu.SemaphoreType.DMA,
        ],
    )(x2d, pos)


def _gmm_body(off_ref, xs_ref, w_ref, b_ref, out_ref):
    e = pl.program_id(0)
    start = off_ref[e]
    end = off_ref[e + 1]
    w0 = start // T
    nwin = (end + T - 1) // T - w0
    wm = w_ref[0]
    bias = b_ref[0]

    def body(k, _):
        r0 = (w0 + k) * T
        x = xs_ref[pl.ds(r0, T), :]
        y = lax.dot_general(x, wm, (((1,), (1,)), ((), ())),
                            preferred_element_type=jnp.float32) + bias
        r = lax.broadcasted_iota(jnp.int32, (T, 1), 0) + r0
        mask = (r >= start) & (r < end)
        out_ref[pl.ds(r0, T), :] = jnp.where(mask, y, out_ref[pl.ds(r0, T), :])
        return 0

    lax.fori_loop(0, nwin, body, 0)


def _grouped_matmul(xs, expert_W, expert_b, off):
    grid_spec = pltpu.PrefetchScalarGridSpec(
        num_scalar_prefetch=1,
        grid=(E,),
        in_specs=[
            pl.BlockSpec((A, H), lambda e, off: (0, 0)),
            pl.BlockSpec((1, H, H), lambda e, off: (e, 0, 0)),
            pl.BlockSpec((1, 1, H), lambda e, off: (e, 0, 0)),
        ],
        out_specs=pl.BlockSpec((A, H), lambda e, off: (0, 0)),
    )
    return pl.pallas_call(
        _gmm_body,
        grid_spec=grid_spec,
        out_shape=jax.ShapeDtypeStruct((A, H), jnp.float32),
    )(off, xs, expert_W, expert_b.reshape(E, 1, H))


def _combine_body(y_hbm, p1_hbm, p2_hbm, out_hbm,
                  i1_v, i2_v, y1_v, y2_v, sem1, sem2):
    wid = lax.axis_index("s") * NC + lax.axis_index("c")
    rows = S // NW
    base = wid * rows
    pltpu.sync_copy(p1_hbm.at[pl.ds(base, rows)], i1_v)
    pltpu.sync_copy(p2_hbm.at[pl.ds(base, rows)], i2_v)
    c1 = pltpu.async_copy(y_hbm.at[i1_v], y1_v, sem1)
    c2 = pltpu.async_copy(y_hbm.at[i2_v], y2_v, sem2)
    c1.wait()
    c2.wait()

    def addrow(r, _):
        for c in range(0, H, 16):
            y1_v[r, pl.ds(c, 16)] = (y1_v[r, pl.ds(c, 16)]
                                     + y2_v[r, pl.ds(c, 16)])
        return 0

    lax.fori_loop(0, rows, addrow, 0)
    pltpu.sync_copy(y1_v, out_hbm.at[pl.ds(base, rows)])


def _combine(ys, p1, p2):
    mesh = plsc.VectorSubcoreMesh(core_axis_name="c", subcore_axis_name="s",
                                  num_cores=NC, num_subcores=NS)
    rows = S // NW
    return pl.kernel(
        _combine_body,
        out_type=jax.ShapeDtypeStruct((S, H), jnp.float32),
        mesh=mesh,
        scratch_types=[
            pltpu.VMEM((rows,), jnp.int32),
            pltpu.VMEM((rows,), jnp.int32),
            pltpu.VMEM((rows, H), jnp.float32),
            pltpu.VMEM((rows, H), jnp.float32),
            pltpu.SemaphoreType.DMA,
            pltpu.SemaphoreType.DMA,
        ],
    )(ys, p1, p2)


@jax.jit
def kernel(hidden_states, router_W, router_b, expert_W, expert_b):
    x2d = hidden_states.reshape(S, H)

    posr, offr = _route(x2d, router_W, router_b)
    pos = posr.reshape(A)
    p1 = pos[:S]
    p2 = pos[S:]
    off = jnp.concatenate(
        [offr.reshape(E), jnp.full((1,), A, jnp.int32)])

    tok = jnp.arange(S, dtype=jnp.int32)
    sorted_tok = jnp.zeros((A,), jnp.int32).at[pos].set(
        jnp.concatenate([tok, tok]))
    xs = _scatter_sorted(x2d, sorted_tok)
    ys = _grouped_matmul(xs, expert_W, expert_b, off)
    out2d = _combine(ys, p1, p2)
    return out2d.reshape(1, S, H)
